# baseline (speedup 1.0000x reference)
import jax
import jax.numpy as jnp
from jax.experimental import pallas as pl
from jax.experimental.pallas import tpu as pltpu

N = 320000
D = 128
K = 32
INTRA_MARGIN = 0.5
INTER_MARGIN = 1.5
INTRA_W = 1.0
INTER_W = 1.0
REG_W = 0.001

BN = 40000
NB = N // BN
CH = 2000
NCH = BN // CH


def _mm(a, b, dims):
    return jax.lax.dot_general(
        a, b, (dims, ((), ())), preferred_element_type=jnp.float32
    )


def _onehot_t(lab):
    return (
        lab == jax.lax.broadcasted_iota(lab.dtype, (K, 1), 0)
    ).astype(jnp.float32)


def _disc_loss_kernel(lab_ref, f_ref, out_ref, sums_ref, counts_ref, intra_ref):
    p = pl.program_id(0)
    i = pl.program_id(1)

    @pl.when(jnp.logical_and(p == 0, i == 0))
    def _init():
        sums_ref[...] = jnp.zeros_like(sums_ref)
        counts_ref[...] = jnp.zeros_like(counts_ref)
        intra_ref[...] = jnp.zeros_like(intra_ref)

    @pl.when(p == 0)
    def _phase0():
        for j in range(NCH):
            f = f_ref[j * CH:(j + 1) * CH, :]
            oh_t = _onehot_t(lab_ref[0, j, :, :])
            sums_ref[...] += _mm(oh_t, f, ((1,), (0,)))
            counts_ref[...] += _mm(
                oh_t, jnp.ones((CH, 1), jnp.float32), ((1,), (0,))
            )

    @pl.when(jnp.logical_and(p == 1, i == 0))
    def _means():
        sums_ref[...] = sums_ref[...] / counts_ref[...]

    @pl.when(p == 1)
    def _phase1():
        means = sums_ref[...]
        c = means - 1e-08
        csq_row = jnp.sum(c * c, axis=1)[None, :]
        for j in range(NCH):
            f = f_ref[j * CH:(j + 1) * CH, :]
            oh_t = _onehot_t(lab_ref[0, j, :, :])
            f2 = f * f
            dots_t = _mm(c, f, ((1,), (1,)))
            q_t = _mm(jnp.ones((1, D), jnp.float32), f2, ((1,), (1,)))
            seldot = _mm(
                jnp.ones((1, K), jnp.float32), oh_t * dots_t, ((1,), (0,))
            )
            selcsq = _mm(csq_row, oh_t, ((1,), (0,)))
            dist2 = q_t - 2.0 * seldot + selcsq
            dist = jnp.sqrt(dist2)
            hinge = jnp.maximum(dist - INTRA_MARGIN, 0.0)
            h2m_t = oh_t * (hinge * hinge)
            intra_ref[...] += _mm(
                h2m_t, jnp.ones((CH, 1), jnp.float32), ((1,), (0,))
            )

        @pl.when(i == NB - 1)
        def _finish():
            intra_loss = (
                jnp.sum(intra_ref[:, 0] / counts_ref[:, 0]) / K
            )

            md = means[:, None, :] - means[None, :, :] + 1e-08
            pair_dist = jnp.sqrt(jnp.sum(md * md, axis=-1))
            pair_hinge = jnp.maximum(2.0 * INTER_MARGIN - pair_dist, 0.0)
            offdiag = 1.0 - jnp.eye(K, dtype=jnp.float32)
            inter_loss = jnp.sum(pair_hinge * pair_hinge * offdiag) / float(
                (K - 1) * K
            )

            mr = means + 1e-08
            reg_loss = jnp.sum(jnp.sqrt(jnp.sum(mr * mr, axis=1))) / float(K)

            loss = (
                INTRA_W * intra_loss + INTER_W * inter_loss + REG_W * reg_loss
            )
            out_ref[...] = jnp.broadcast_to(loss, (1, 1))


@jax.jit
def kernel(features, labels):
    labels4 = labels.astype(jnp.int32).reshape(NB, NCH, 1, CH)
    out = pl.pallas_call(
        _disc_loss_kernel,
        grid=(2, NB),
        in_specs=[
            pl.BlockSpec((1, NCH, 1, CH), lambda p, i: (i, 0, 0, 0)),
            pl.BlockSpec((BN, D), lambda p, i: (i, 0)),
        ],
        out_specs=pl.BlockSpec((1, 1), lambda p, i: (0, 0)),
        out_shape=jax.ShapeDtypeStruct((1, 1), jnp.float32),
        scratch_shapes=[
            pltpu.VMEM((K, D), jnp.float32),
            pltpu.VMEM((K, 1), jnp.float32),
            pltpu.VMEM((K, 1), jnp.float32),
        ],
    )(labels4, features)
    return out.reshape(())


# speedup vs baseline: 1.1048x; 1.1048x over previous
import jax
import jax.numpy as jnp
from jax.experimental import pallas as pl
from jax.experimental.pallas import tpu as pltpu

N = 320000
D = 128
K = 32
INTRA_MARGIN = 0.5
INTER_MARGIN = 1.5
INTRA_W = 1.0
INTER_W = 1.0
REG_W = 0.001

BN = 32000
NB = N // BN
CH = 4000
NCH = BN // CH


def _mm(a, b, dims):
    return jax.lax.dot_general(
        a, b, (dims, ((), ())), preferred_element_type=jnp.float32
    )


def _onehot_t(lab):
    return (
        lab == jax.lax.broadcasted_iota(lab.dtype, (K, 1), 0)
    ).astype(jnp.float32)


def _disc_loss_kernel(lab_ref, f_ref, out_ref, sums_ref, counts_ref, intra_ref):
    p = pl.program_id(0)
    i = pl.program_id(1)

    @pl.when(jnp.logical_and(p == 0, i == 0))
    def _init():
        sums_ref[...] = jnp.zeros_like(sums_ref)
        counts_ref[...] = jnp.zeros_like(counts_ref)
        intra_ref[...] = jnp.zeros_like(intra_ref)

    @pl.when(p == 0)
    def _phase0():
        for j in range(NCH):
            f = f_ref[j * CH:(j + 1) * CH, :]
            oh_t = _onehot_t(lab_ref[0, j, :, :])
            sums_ref[...] += _mm(oh_t, f, ((1,), (0,)))
            counts_ref[...] += _mm(
                oh_t, jnp.ones((CH, 1), jnp.float32), ((1,), (0,))
            )

    @pl.when(jnp.logical_and(p == 1, i == 0))
    def _means():
        sums_ref[...] = sums_ref[...] / counts_ref[...]

    @pl.when(p == 1)
    def _phase1():
        means = sums_ref[...]
        c = means - 1e-08
        csq_row = jnp.sum(c * c, axis=1)[None, :]
        for j in range(NCH):
            f = f_ref[j * CH:(j + 1) * CH, :]
            oh_t = _onehot_t(lab_ref[0, j, :, :])
            f2 = f * f
            dots_t = _mm(c, f, ((1,), (1,)))
            q_t = _mm(jnp.ones((1, D), jnp.float32), f2, ((1,), (1,)))
            seldot = _mm(
                jnp.ones((1, K), jnp.float32), oh_t * dots_t, ((1,), (0,))
            )
            selcsq = _mm(csq_row, oh_t, ((1,), (0,)))
            dist2 = q_t - 2.0 * seldot + selcsq
            dist = jnp.sqrt(dist2)
            hinge = jnp.maximum(dist - INTRA_MARGIN, 0.0)
            h2m_t = oh_t * (hinge * hinge)
            intra_ref[...] += _mm(
                h2m_t, jnp.ones((CH, 1), jnp.float32), ((1,), (0,))
            )

        @pl.when(i == NB - 1)
        def _finish():
            intra_loss = (
                jnp.sum(intra_ref[:, 0] / counts_ref[:, 0]) / K
            )

            md = means[:, None, :] - means[None, :, :] + 1e-08
            pair_dist = jnp.sqrt(jnp.sum(md * md, axis=-1))
            pair_hinge = jnp.maximum(2.0 * INTER_MARGIN - pair_dist, 0.0)
            offdiag = 1.0 - jnp.eye(K, dtype=jnp.float32)
            inter_loss = jnp.sum(pair_hinge * pair_hinge * offdiag) / float(
                (K - 1) * K
            )

            mr = means + 1e-08
            reg_loss = jnp.sum(jnp.sqrt(jnp.sum(mr * mr, axis=1))) / float(K)

            loss = (
                INTRA_W * intra_loss + INTER_W * inter_loss + REG_W * reg_loss
            )
            out_ref[...] = jnp.broadcast_to(loss, (1, 1))


@jax.jit
def kernel(features, labels):
    labels4 = labels.astype(jnp.int32).reshape(NB, NCH, 1, CH)
    out = pl.pallas_call(
        _disc_loss_kernel,
        grid=(2, NB),
        in_specs=[
            pl.BlockSpec((1, NCH, 1, CH), lambda p, i: (i, 0, 0, 0)),
            pl.BlockSpec((BN, D), lambda p, i: (i, 0)),
        ],
        out_specs=pl.BlockSpec((1, 1), lambda p, i: (0, 0)),
        out_shape=jax.ShapeDtypeStruct((1, 1), jnp.float32),
        scratch_shapes=[
            pltpu.VMEM((K, D), jnp.float32),
            pltpu.VMEM((K, 1), jnp.float32),
            pltpu.VMEM((K, 1), jnp.float32),
        ],
    )(labels4, features)
    return out.reshape(())


# speedup vs baseline: 1.1153x; 1.0095x over previous
import jax
import jax.numpy as jnp
from jax.experimental import pallas as pl
from jax.experimental.pallas import tpu as pltpu

N = 320000
D = 128
K = 32
INTRA_MARGIN = 0.5
INTER_MARGIN = 1.5
INTRA_W = 1.0
INTER_W = 1.0
REG_W = 0.001

BN = 32000
NB = N // BN
CH = 8000
NCH = BN // CH


def _mm(a, b, dims):
    return jax.lax.dot_general(
        a, b, (dims, ((), ())), preferred_element_type=jnp.float32
    )


def _onehot_t(lab):
    return (
        lab == jax.lax.broadcasted_iota(lab.dtype, (K, 1), 0)
    ).astype(jnp.float32)


def _disc_loss_kernel(lab_ref, f_ref, out_ref, sums_ref, counts_ref, intra_ref):
    p = pl.program_id(0)
    i = pl.program_id(1)

    @pl.when(jnp.logical_and(p == 0, i == 0))
    def _init():
        sums_ref[...] = jnp.zeros_like(sums_ref)
        counts_ref[...] = jnp.zeros_like(counts_ref)
        intra_ref[...] = jnp.zeros_like(intra_ref)

    @pl.when(p == 0)
    def _phase0():
        for j in range(NCH):
            f = f_ref[j * CH:(j + 1) * CH, :]
            oh_t = _onehot_t(lab_ref[0, j, :, :])
            sums_ref[...] += _mm(oh_t, f, ((1,), (0,)))
            counts_ref[...] += _mm(
                oh_t, jnp.ones((CH, 1), jnp.float32), ((1,), (0,))
            )

    @pl.when(jnp.logical_and(p == 1, i == 0))
    def _means():
        sums_ref[...] = sums_ref[...] / counts_ref[...]

    @pl.when(p == 1)
    def _phase1():
        means = sums_ref[...]
        c = means - 1e-08
        csq_row = jnp.sum(c * c, axis=1)[None, :]
        for j in range(NCH):
            f = f_ref[j * CH:(j + 1) * CH, :]
            oh_t = _onehot_t(lab_ref[0, j, :, :])
            f2 = f * f
            dots_t = _mm(c, f, ((1,), (1,)))
            q_t = _mm(jnp.ones((1, D), jnp.float32), f2, ((1,), (1,)))
            seldot = _mm(
                jnp.ones((1, K), jnp.float32), oh_t * dots_t, ((1,), (0,))
            )
            selcsq = _mm(csq_row, oh_t, ((1,), (0,)))
            dist2 = q_t - 2.0 * seldot + selcsq
            dist = jnp.sqrt(dist2)
            hinge = jnp.maximum(dist - INTRA_MARGIN, 0.0)
            h2m_t = oh_t * (hinge * hinge)
            intra_ref[...] += _mm(
                h2m_t, jnp.ones((CH, 1), jnp.float32), ((1,), (0,))
            )

        @pl.when(i == NB - 1)
        def _finish():
            intra_loss = (
                jnp.sum(intra_ref[:, 0] / counts_ref[:, 0]) / K
            )

            md = means[:, None, :] - means[None, :, :] + 1e-08
            pair_dist = jnp.sqrt(jnp.sum(md * md, axis=-1))
            pair_hinge = jnp.maximum(2.0 * INTER_MARGIN - pair_dist, 0.0)
            offdiag = 1.0 - jnp.eye(K, dtype=jnp.float32)
            inter_loss = jnp.sum(pair_hinge * pair_hinge * offdiag) / float(
                (K - 1) * K
            )

            mr = means + 1e-08
            reg_loss = jnp.sum(jnp.sqrt(jnp.sum(mr * mr, axis=1))) / float(K)

            loss = (
                INTRA_W * intra_loss + INTER_W * inter_loss + REG_W * reg_loss
            )
            out_ref[...] = jnp.broadcast_to(loss, (1, 1))


@jax.jit
def kernel(features, labels):
    labels4 = labels.astype(jnp.int32).reshape(NB, NCH, 1, CH)
    out = pl.pallas_call(
        _disc_loss_kernel,
        grid=(2, NB),
        in_specs=[
            pl.BlockSpec((1, NCH, 1, CH), lambda p, i: (i, 0, 0, 0)),
            pl.BlockSpec((BN, D), lambda p, i: (i, 0)),
        ],
        out_specs=pl.BlockSpec((1, 1), lambda p, i: (0, 0)),
        out_shape=jax.ShapeDtypeStruct((1, 1), jnp.float32),
        scratch_shapes=[
            pltpu.VMEM((K, D), jnp.float32),
            pltpu.VMEM((K, 1), jnp.float32),
            pltpu.VMEM((K, 1), jnp.float32),
        ],
    )(labels4, features)
    return out.reshape(())
